# baseline (speedup 1.0000x reference)
import functools

import jax
import jax.numpy as jnp
from jax import lax
from jax.experimental import pallas as pl
from jax.experimental.pallas import tpu as pltpu
from jax.experimental.pallas import tpu_sc as plsc

VOCAB = 100000
EMBED_DIM = 64
BATCH = 4096
SEQ_LEN = 200

NUM_CORES = 2
NUM_SUBCORES = 16
NUM_WORKERS = NUM_CORES * NUM_SUBCORES

LANES = 16
BT = BATCH // 128
LT = SEQ_LEN // 8
DT = EMBED_DIM // 8
LG = 4
NU = SEQ_LEN // LG


def _embed_body(x4_hbm, tok_hbm, pos_hbm, out_hbm,
                idxall, grows0, grows1, wbuf0, wbuf1, pos_v,
                gsem0, gsem1, wsem0, wsem1):
  cid = lax.axis_index("c")
  sid = lax.axis_index("s")
  w = sid * NUM_CORES + cid
  grows = (grows0, grows1)
  wbuf = (wbuf0, wbuf1)
  gsem = (gsem0, gsem1)
  wsem = (wsem0, wsem1)

  pltpu.sync_copy(pos_hbm, pos_v)
  pltpu.sync_copy(x4_hbm.at[:, w], idxall)

  iota = lax.broadcasted_iota(jnp.int32, (LANES,), 0)
  dvec = [iota + m * LANES for m in range(EMBED_DIM // LANES)]

  def launch_gather(u, g):
    pltpu.async_copy(
        tok_hbm.at[idxall.at[u // 2, pl.ds((u % 2) * LG * 128, LG * 128)]],
        grows[g], gsem[g])

  launch_gather(0, 0)

  def unit(u, g):
    ng = 1 - g
    un = u + 1

    pltpu.make_async_copy(
        tok_hbm.at[idxall.at[0, pl.ds(0, LG * 128)]], grows[g],
        gsem[g]).wait()

    @pl.when(un < NU)
    def _():
      launch_gather(un, ng)

    for j in range(LG):
      l = u * LG + j
      wb = j % 2
      base = l * EMBED_DIM

      @pl.when(l >= 2)
      def _():
        for t in range(DT):
          pltpu.make_async_copy(
              wbuf[wb].at[pl.ds(t * 8, 8), pl.ds(0, 128)],
              out_hbm.at[0, 0, w], wsem[wb]).wait()

      pv = [pos_v[pl.ds(base + m * LANES, LANES)]
            for m in range(EMBED_DIM // LANES)]

      @pl.loop(0, 128, unroll=4)
      def _bb(bb):
        row = j * 128 + bb
        bbsplat = jnp.full((LANES,), bb, jnp.int32)
        for m in range(EMBED_DIM // LANES):
          gv = grows[g][row, pl.ds(m * LANES, LANES)] + pv[m]
          plsc.store_scatter(wbuf[wb], [dvec[m], bbsplat], gv)

      for t in range(DT):
        pltpu.async_copy(
            wbuf[wb].at[pl.ds(t * 8, 8), pl.ds(0, 128)],
            out_hbm.at[l, t, w], wsem[wb])

  @pl.loop(0, NU, step=2)
  def _u(u):
    unit(u, 0)
    unit(u + 1, 1)

  for b in range(2):
    for t in range(DT):
      pltpu.make_async_copy(
          wbuf[b].at[pl.ds(t * 8, 8), pl.ds(0, 128)],
          out_hbm.at[0, 0, w], wsem[b]).wait()


@jax.jit
def _embed(x4, token_table, pos_flat):
  mesh = plsc.VectorSubcoreMesh(
      core_axis_name="c", subcore_axis_name="s",
      num_cores=NUM_CORES, num_subcores=NUM_SUBCORES,
  )
  run = pl.kernel(
      _embed_body,
      out_type=jax.ShapeDtypeStruct(
          (SEQ_LEN, DT, BT, 8, 128), jnp.float32),
      mesh=mesh,
      compiler_params=pltpu.CompilerParams(
          use_tc_tiling_on_sc=False, needs_layout_passes=False),
      scratch_types=[
          pltpu.VMEM((LT, 8 * 128), jnp.int32),
          pltpu.VMEM((LG * 128, EMBED_DIM), jnp.float32),
          pltpu.VMEM((LG * 128, EMBED_DIM), jnp.float32),
          pltpu.VMEM((EMBED_DIM, 129), jnp.float32),
          pltpu.VMEM((EMBED_DIM, 129), jnp.float32),
          pltpu.VMEM((SEQ_LEN * EMBED_DIM,), jnp.float32),
          pltpu.SemaphoreType.DMA,
          pltpu.SemaphoreType.DMA,
          pltpu.SemaphoreType.DMA,
          pltpu.SemaphoreType.DMA,
      ],
  )
  return run(x4, token_table, pos_flat)


def kernel(x, token_table, pos_table):
  x4 = (x.astype(jnp.int32).T
        .reshape(LT, 8, BT, 128).transpose(0, 2, 1, 3)
        .reshape(LT, BT, 8 * 128))
  out5 = _embed(x4, token_table, pos_table.reshape(-1))
  return out5.transpose(2, 4, 0, 1, 3).reshape(BATCH, SEQ_LEN, EMBED_DIM)


# speedup vs baseline: 1.0078x; 1.0078x over previous
import functools

import jax
import jax.numpy as jnp
from jax import lax
from jax.experimental import pallas as pl
from jax.experimental.pallas import tpu as pltpu
from jax.experimental.pallas import tpu_sc as plsc

VOCAB = 100000
EMBED_DIM = 64
BATCH = 4096
SEQ_LEN = 200

NUM_CORES = 2
NUM_SUBCORES = 16
NUM_WORKERS = NUM_CORES * NUM_SUBCORES

LANES = 16
BT = BATCH // 128
LT = SEQ_LEN // 8
DT = EMBED_DIM // 8
LG = 4
NU = SEQ_LEN // LG


def _embed_body(x4_hbm, tok_hbm, pos_hbm, out_hbm,
                idxall, grows0, grows1, wbuf0, wbuf1, pos_v,
                gsem0, gsem1, wsem0, wsem1):
  cid = lax.axis_index("c")
  sid = lax.axis_index("s")
  w = sid * NUM_CORES + cid
  grows = (grows0, grows1)
  wbuf = (wbuf0, wbuf1)
  gsem = (gsem0, gsem1)
  wsem = (wsem0, wsem1)

  pltpu.sync_copy(pos_hbm, pos_v)
  pltpu.sync_copy(x4_hbm.at[:, w], idxall)

  iota = lax.broadcasted_iota(jnp.int32, (LANES,), 0)
  tidx = [(iota + m * LANES) // 8 for m in range(EMBED_DIM // LANES)]
  sidx = [(iota + m * LANES) % 8 for m in range(EMBED_DIM // LANES)]

  def launch_gather(u, g):
    pltpu.async_copy(
        tok_hbm.at[idxall.at[u // 2, pl.ds((u % 2) * LG * 128, LG * 128)]],
        grows[g], gsem[g])

  launch_gather(0, 0)

  def unit(u, g):
    ng = 1 - g
    un = u + 1

    pltpu.make_async_copy(
        tok_hbm.at[idxall.at[0, pl.ds(0, LG * 128)]], grows[g],
        gsem[g]).wait()

    @pl.when(un < NU)
    def _():
      launch_gather(un, ng)

    for j in range(LG):
      l = u * LG + j
      wb = j % 2
      base = l * EMBED_DIM

      @pl.when(l >= 2)
      def _():
        pltpu.make_async_copy(
            wbuf[wb].at[:, :, pl.ds(0, 128)],
            out_hbm.at[0, :, w], wsem[wb]).wait()

      pv = [pos_v[pl.ds(base + m * LANES, LANES)]
            for m in range(EMBED_DIM // LANES)]

      @pl.loop(0, 128, unroll=2)
      def _bb(bb):
        row = j * 128 + bb
        bbsplat = jnp.full((LANES,), bb, jnp.int32)
        for m in range(EMBED_DIM // LANES):
          gv = grows[g][row, pl.ds(m * LANES, LANES)] + pv[m]
          plsc.store_scatter(wbuf[wb], [tidx[m], sidx[m], bbsplat], gv)

      pltpu.async_copy(
          wbuf[wb].at[:, :, pl.ds(0, 128)], out_hbm.at[l, :, w], wsem[wb])

  @pl.loop(0, NU, step=2)
  def _u(u):
    unit(u, 0)
    unit(u + 1, 1)

  for b in range(2):
    pltpu.make_async_copy(
        wbuf[b].at[:, :, pl.ds(0, 128)], out_hbm.at[0, :, w], wsem[b]).wait()


@jax.jit
def _embed(x4, token_table, pos_flat):
  mesh = plsc.VectorSubcoreMesh(
      core_axis_name="c", subcore_axis_name="s",
      num_cores=NUM_CORES, num_subcores=NUM_SUBCORES,
  )
  run = pl.kernel(
      _embed_body,
      out_type=jax.ShapeDtypeStruct(
          (SEQ_LEN, DT, BT, 8, 128), jnp.float32),
      mesh=mesh,
      compiler_params=pltpu.CompilerParams(
          use_tc_tiling_on_sc=False, needs_layout_passes=False),
      scratch_types=[
          pltpu.VMEM((LT, 8 * 128), jnp.int32),
          pltpu.VMEM((LG * 128, EMBED_DIM), jnp.float32),
          pltpu.VMEM((LG * 128, EMBED_DIM), jnp.float32),
          pltpu.VMEM((DT, 8, 129), jnp.float32),
          pltpu.VMEM((DT, 8, 129), jnp.float32),
          pltpu.VMEM((SEQ_LEN * EMBED_DIM,), jnp.float32),
          pltpu.SemaphoreType.DMA,
          pltpu.SemaphoreType.DMA,
          pltpu.SemaphoreType.DMA,
          pltpu.SemaphoreType.DMA,
      ],
  )
  return run(x4, token_table, pos_flat)


def kernel(x, token_table, pos_table):
  x4 = (x.astype(jnp.int32).T
        .reshape(LT, 8, BT, 128).transpose(0, 2, 1, 3)
        .reshape(LT, BT, 8 * 128))
  out5 = _embed(x4, token_table, pos_table.reshape(-1))
  return out5.transpose(2, 4, 0, 1, 3).reshape(BATCH, SEQ_LEN, EMBED_DIM)


# speedup vs baseline: 1.0109x; 1.0030x over previous
import functools

import jax
import jax.numpy as jnp
from jax import lax
from jax.experimental import pallas as pl
from jax.experimental.pallas import tpu as pltpu
from jax.experimental.pallas import tpu_sc as plsc

VOCAB = 100000
EMBED_DIM = 64
BATCH = 4096
SEQ_LEN = 200

NUM_CORES = 2
NUM_SUBCORES = 16
NUM_WORKERS = NUM_CORES * NUM_SUBCORES

LANES = 16
BT = BATCH // 128
LT = SEQ_LEN // 8
DT = EMBED_DIM // 8
LG = 4
NU = SEQ_LEN // LG


def _embed_body(x4_hbm, tok_hbm, pos_hbm, out_hbm,
                idxall, grows0, grows1, wbuf0, wbuf1, pos_v,
                gsem0, gsem1, wsem0, wsem1):
  cid = lax.axis_index("c")
  sid = lax.axis_index("s")
  w = sid * NUM_CORES + cid
  grows = (grows0, grows1)
  wbuf = (wbuf0, wbuf1)
  gsem = (gsem0, gsem1)
  wsem = (wsem0, wsem1)

  pltpu.sync_copy(pos_hbm, pos_v)
  pltpu.sync_copy(x4_hbm.at[:, w], idxall)

  iota = lax.broadcasted_iota(jnp.int32, (LANES,), 0)
  tidx = [(iota + m * LANES) // 8 for m in range(EMBED_DIM // LANES)]
  sidx = [(iota + m * LANES) % 8 for m in range(EMBED_DIM // LANES)]

  def launch_gather(u, g):
    pltpu.async_copy(
        tok_hbm.at[idxall.at[u // 2, pl.ds((u % 2) * LG * 128, LG * 128)]],
        grows[g], gsem[g])

  launch_gather(0, 0)

  def unit(u, g):
    ng = 1 - g
    un = u + 1

    pltpu.make_async_copy(
        tok_hbm.at[idxall.at[0, pl.ds(0, LG * 128)]], grows[g],
        gsem[g]).wait()

    @pl.when(un < NU)
    def _():
      launch_gather(un, ng)

    for j in range(LG):
      l = u * LG + j
      wb = j % 2
      base = l * EMBED_DIM

      @pl.when(l >= 2)
      def _():
        pltpu.make_async_copy(
            wbuf[wb].at[:, :, pl.ds(0, 128)],
            out_hbm.at[0, :, w], wsem[wb]).wait()

      pv = [pos_v[pl.ds(base + m * LANES, LANES)]
            for m in range(EMBED_DIM // LANES)]

      @pl.loop(0, 128, unroll=4)
      def _bb(bb):
        row = j * 128 + bb
        bbsplat = jnp.full((LANES,), bb, jnp.int32)
        for m in range(EMBED_DIM // LANES):
          gv = grows[g][row, pl.ds(m * LANES, LANES)] + pv[m]
          plsc.store_scatter(wbuf[wb], [tidx[m], sidx[m], bbsplat], gv)

      pltpu.async_copy(
          wbuf[wb].at[:, :, pl.ds(0, 128)], out_hbm.at[l, :, w], wsem[wb])

  @pl.loop(0, NU, step=2)
  def _u(u):
    unit(u, 0)
    unit(u + 1, 1)

  for b in range(2):
    pltpu.make_async_copy(
        wbuf[b].at[:, :, pl.ds(0, 128)], out_hbm.at[0, :, w], wsem[b]).wait()


@jax.jit
def _embed(x4, token_table, pos_flat):
  mesh = plsc.VectorSubcoreMesh(
      core_axis_name="c", subcore_axis_name="s",
      num_cores=NUM_CORES, num_subcores=NUM_SUBCORES,
  )
  run = pl.kernel(
      _embed_body,
      out_type=jax.ShapeDtypeStruct(
          (SEQ_LEN, DT, BT, 8, 128), jnp.float32),
      mesh=mesh,
      compiler_params=pltpu.CompilerParams(
          use_tc_tiling_on_sc=False, needs_layout_passes=False),
      scratch_types=[
          pltpu.VMEM((LT, 8 * 128), jnp.int32),
          pltpu.VMEM((LG * 128, EMBED_DIM), jnp.float32),
          pltpu.VMEM((LG * 128, EMBED_DIM), jnp.float32),
          pltpu.VMEM((DT, 8, 129), jnp.float32),
          pltpu.VMEM((DT, 8, 129), jnp.float32),
          pltpu.VMEM((SEQ_LEN * EMBED_DIM,), jnp.float32),
          pltpu.SemaphoreType.DMA,
          pltpu.SemaphoreType.DMA,
          pltpu.SemaphoreType.DMA,
          pltpu.SemaphoreType.DMA,
      ],
  )
  return run(x4, token_table, pos_flat)


def kernel(x, token_table, pos_table):
  x4 = (x.astype(jnp.int32).T
        .reshape(LT, 8, BT, 128).transpose(0, 2, 1, 3)
        .reshape(LT, BT, 8 * 128))
  out5 = _embed(x4, token_table, pos_table.reshape(-1))
  return out5.transpose(2, 4, 0, 1, 3).reshape(BATCH, SEQ_LEN, EMBED_DIM)
